# Initial kernel scaffold; baseline (speedup 1.0000x reference)
import jax
import jax.numpy as jnp
from jax.experimental import pallas as pl


def kernel(x, r_ij, f_ij, neighbours, pairwise_mask, W_in2f, Wf, bf, Wout, bout):
    raise NotImplementedError("write your pallas kernel here")


# speedup vs baseline: 3.1588x; 3.1588x over previous
import functools

import jax
import jax.numpy as jnp
from jax.experimental import pallas as pl
from jax.experimental.pallas import tpu as pltpu
from jax.experimental.pallas import tpu_sc as plsc


def _in2f_matmul(x2, w):
    n, d = x2.shape
    f = w.shape[1]
    bm = 1000

    def body(x_ref, w_ref, o_ref):
        o_ref[...] = jnp.dot(x_ref[...], w_ref[...],
                             preferred_element_type=jnp.float32)

    return pl.pallas_call(
        body,
        grid=(n // bm,),
        in_specs=[
            pl.BlockSpec((bm, d), lambda i: (i, 0)),
            pl.BlockSpec((d, f), lambda i: (0, 0)),
        ],
        out_specs=pl.BlockSpec((bm, f), lambda i: (i, 0)),
        out_shape=jax.ShapeDtypeStruct((n, f), jnp.float32),
    )(x2, w)


def _sc_gather(table, idx):
    num_idx = idx.shape[0]
    d = table.shape[1]
    window = 256
    idx2 = idx.reshape(1, num_idx)
    mesh = plsc.VectorSubcoreMesh(core_axis_name="core",
                                  subcore_axis_name="subcore")

    @functools.partial(
        pl.kernel,
        out_type=jax.ShapeDtypeStruct((num_idx, d), table.dtype),
        mesh=mesh,
    )
    def k(table_hbm, i_hbm, o_hbm):
        def body(i_vmem, o_vmem):
            pltpu.sync_copy(table_hbm.at[i_vmem.at[0]], o_vmem)

        pltpu.emit_pipeline(
            body,
            grid=(num_idx // window,),
            in_specs=[pl.BlockSpec((1, window), lambda i: (0, i))],
            out_specs=[pl.BlockSpec((window, d), lambda i: (i, 0))],
            core_axis_name=("core", "subcore"),
            dimension_semantics=(pltpu.PARALLEL,),
        )(i_hbm, o_hbm)

    return k(table, idx2)


def _fused_tail(f_flat, yg, mask, wf, bf2, wout, bout2, na, nnbh):
    nf = wf.shape[1]
    ng = wf.shape[0]
    nout = wout.shape[1]
    ba = 400
    be = ba * nnbh

    def body(f_ref, yg_ref, m_ref, wf_ref, bf_ref, wout_ref, bout_ref, o_ref):
        w = jnp.dot(f_ref[...], wf_ref[...],
                    preferred_element_type=jnp.float32) + bf_ref[...]
        z = yg_ref[...] * w
        z3 = z.reshape(ba, nnbh, nf) * m_ref[...][:, :, None]
        zs = jnp.sum(z3, axis=1)
        o_ref[...] = jnp.dot(zs, wout_ref[...],
                             preferred_element_type=jnp.float32) + bout_ref[...]

    return pl.pallas_call(
        body,
        grid=(na // ba,),
        in_specs=[
            pl.BlockSpec((be, ng), lambda i: (i, 0)),
            pl.BlockSpec((be, nf), lambda i: (i, 0)),
            pl.BlockSpec((ba, nnbh), lambda i: (i, 0)),
            pl.BlockSpec((ng, nf), lambda i: (0, 0)),
            pl.BlockSpec((1, nf), lambda i: (0, 0)),
            pl.BlockSpec((nf, nout), lambda i: (0, 0)),
            pl.BlockSpec((1, nout), lambda i: (0, 0)),
        ],
        out_specs=pl.BlockSpec((ba, nout), lambda i: (i, 0)),
        out_shape=jax.ShapeDtypeStruct((na, nout), jnp.float32),
    )(f_flat, yg, mask, wf, bf2, wout, bout2)


def kernel(x, r_ij, f_ij, neighbours, pairwise_mask, W_in2f, Wf, bf, Wout, bout):
    nb, na, nin = x.shape
    nnbh = neighbours.shape[2]
    ng = f_ij.shape[3]

    x2 = x[0]
    idx = neighbours[0].reshape(-1).astype(jnp.int32)
    f_flat = f_ij[0].reshape(na * nnbh, ng)
    mask = pairwise_mask[0]

    y = _in2f_matmul(x2, W_in2f)
    yg = _sc_gather(y, idx)
    out = _fused_tail(f_flat, yg, mask, Wf, bf.reshape(1, -1), Wout,
                      bout.reshape(1, -1), na, nnbh)
    return out[None]
